# baseline (speedup 1.0000x reference)
import jax
import jax.numpy as jnp
from jax.experimental import pallas as pl

_THRESHOLD = 0.05


def _stats_kernel(x_ref, nd_ref, acc_ref):
    @pl.when(pl.program_id(0) == 0)
    def _init():
        acc_ref[:, :] = jnp.zeros_like(acc_ref)

    x = x_ref[:, :]
    n = nd_ref[:, 0:3]
    d = nd_ref[:, 3:4]
    xt = jax.lax.dot_general(jnp.eye(3, dtype=jnp.float32), x,
                             (((1,), (1,)), ((), ())),
                             preferred_element_type=jnp.float32)
    dots = jax.lax.dot_general(n, xt, (((1,), (0,)), ((), ())),
                               preferred_element_type=jnp.float32)
    mask = (jnp.abs(dots - d) < _THRESHOLD).astype(jnp.float32)
    x0, x1, x2 = xt[0:1, :], xt[1:2, :], xt[2:3, :]
    z = jnp.concatenate(
        [jnp.ones_like(x0), x0, x1, x2,
         x0 * x0, x0 * x1, x0 * x2, x1 * x1, x1 * x2, x2 * x2,
         jnp.zeros((6, x0.shape[1]), jnp.float32)], axis=0)
    acc_ref[:, :] += jax.lax.dot_general(
        mask, z, (((1,), (1,)), ((), ())),
        preferred_element_type=jnp.float32)


def _proj_kernel(x_ref, nd_ref, rf_ref, out_ref):
    x = x_ref[:, :]
    n = nd_ref[:, 0:3]
    d = nd_ref[:, 3:4]
    xt = jax.lax.dot_general(jnp.eye(3, dtype=jnp.float32), x,
                             (((1,), (1,)), ((), ())),
                             preferred_element_type=jnp.float32)
    dots = jax.lax.dot_general(n, xt, (((1,), (0,)), ((), ())),
                               preferred_element_type=jnp.float32)
    mask = (jnp.abs(dots - d) < _THRESHOLD).astype(jnp.float32)
    p0, p1, p2 = xt[0:1, :], xt[1:2, :], xt[2:3, :]
    ts = []
    for i in range(16):
        m = mask[i:i + 1, :] * rf_ref[i:i + 1, 4:5]
        r0 = rf_ref[i:i + 1, 0:1]
        r1 = rf_ref[i:i + 1, 1:2]
        r2 = rf_ref[i:i + 1, 2:3]
        rd = rf_ref[i:i + 1, 3:4]
        t = (r0 * p0 + r1 * p1 + r2 * p2 - rd) * m
        ts.append(t)
        p0 = p0 - r0 * t
        p1 = p1 - r1 * t
        p2 = p2 - r2 * t
    tmat = jnp.concatenate(ts, axis=0)
    rn = rf_ref[:, 0:3]
    upd = jax.lax.dot_general(tmat, rn, (((0,), (0,)), ((), ())),
                              preferred_element_type=jnp.float32)
    out_ref[:, :] = x - upd


def kernel(points, normals, distances):
    n_pts = points.shape[0]
    block = 16000 if n_pts % 16000 == 0 else n_pts
    grid = n_pts // block

    nd = jnp.concatenate([normals, distances[:, None]], axis=1)

    acc = pl.pallas_call(
        _stats_kernel,
        grid=(grid,),
        in_specs=[pl.BlockSpec((block, 3), lambda i: (i, 0)),
                  pl.BlockSpec((16, 4), lambda i: (0, 0))],
        out_specs=pl.BlockSpec((16, 16), lambda i: (0, 0)),
        out_shape=jax.ShapeDtypeStruct((16, 16), jnp.float32),
    )(points, nd)

    cnt = acc[:, 0]
    s1 = acc[:, 1:4]
    s2 = jnp.stack([acc[:, 4], acc[:, 5], acc[:, 6],
                    acc[:, 5], acc[:, 7], acc[:, 8],
                    acc[:, 6], acc[:, 8], acc[:, 9]], axis=1).reshape(16, 3, 3)
    denom = jnp.maximum(cnt, 1.0)
    cov = s2 - (s1[:, :, None] * s1[:, None, :]) / denom[:, None, None]
    _, _, vh = jnp.linalg.svd(cov)
    rn = vh[:, 2, :]
    sign = jnp.where(jnp.sum(rn * normals, axis=1) < 0, -1.0, 1.0)
    rn = rn * sign[:, None]
    centroid = s1 / denom[:, None]
    rd = jnp.sum(centroid * rn, axis=1)
    gate = (cnt >= 3.0).astype(jnp.float32)
    rf = jnp.concatenate(
        [rn, rd[:, None], gate[:, None], jnp.zeros((16, 3), jnp.float32)],
        axis=1)

    return pl.pallas_call(
        _proj_kernel,
        grid=(grid,),
        in_specs=[pl.BlockSpec((block, 3), lambda i: (i, 0)),
                  pl.BlockSpec((16, 4), lambda i: (0, 0)),
                  pl.BlockSpec((16, 8), lambda i: (0, 0))],
        out_specs=pl.BlockSpec((block, 3), lambda i: (i, 0)),
        out_shape=jax.ShapeDtypeStruct((n_pts, 3), jnp.float32),
    )(points, nd, rf)


# speedup vs baseline: 2.6698x; 2.6698x over previous
import jax
import jax.numpy as jnp
from jax.experimental import pallas as pl

_THRESHOLD = 0.05


def _stats_kernel(xt_ref, nd_ref, acc_ref):
    @pl.when(pl.program_id(0) == 0)
    def _init():
        acc_ref[:, :] = jnp.zeros_like(acc_ref)

    x = xt_ref[:, :]
    n = nd_ref[:, 0:3]
    d = nd_ref[:, 3:4]
    dots = jax.lax.dot_general(n, x, (((1,), (0,)), ((), ())),
                               preferred_element_type=jnp.float32)
    mask = (jnp.abs(dots - d) < _THRESHOLD).astype(jnp.float32)
    x0, x1, x2 = x[0:1, :], x[1:2, :], x[2:3, :]
    z = jnp.concatenate(
        [jnp.ones_like(x0), x0, x1, x2,
         x0 * x0, x0 * x1, x0 * x2, x1 * x1, x1 * x2, x2 * x2,
         jnp.zeros((6, x0.shape[1]), jnp.float32)], axis=0)
    acc_ref[:, :] += jax.lax.dot_general(
        mask, z, (((1,), (1,)), ((), ())),
        preferred_element_type=jnp.float32)


def _proj_kernel(xt_ref, nd_ref, rf_ref, out_ref):
    x = xt_ref[:, :]
    n = nd_ref[:, 0:3]
    d = nd_ref[:, 3:4]
    dots = jax.lax.dot_general(n, x, (((1,), (0,)), ((), ())),
                               preferred_element_type=jnp.float32)
    mask = (jnp.abs(dots - d) < _THRESHOLD).astype(jnp.float32)
    p0, p1, p2 = x[0:1, :], x[1:2, :], x[2:3, :]
    for i in range(16):
        m = mask[i:i + 1, :] * rf_ref[i:i + 1, 4:5]
        r0 = rf_ref[i:i + 1, 0:1]
        r1 = rf_ref[i:i + 1, 1:2]
        r2 = rf_ref[i:i + 1, 2:3]
        rd = rf_ref[i:i + 1, 3:4]
        t = (r0 * p0 + r1 * p1 + r2 * p2 - rd) * m
        p0 = p0 - r0 * t
        p1 = p1 - r1 * t
        p2 = p2 - r2 * t
    out_ref[0:1, :] = p0
    out_ref[1:2, :] = p1
    out_ref[2:3, :] = p2


def kernel(points, normals, distances):
    n_pts = points.shape[0]
    block = 80000 if n_pts % 80000 == 0 else n_pts
    grid = n_pts // block

    xt = points.T
    nd = jnp.concatenate([normals, distances[:, None]], axis=1)

    acc = pl.pallas_call(
        _stats_kernel,
        grid=(grid,),
        in_specs=[pl.BlockSpec((3, block), lambda i: (0, i)),
                  pl.BlockSpec((16, 4), lambda i: (0, 0))],
        out_specs=pl.BlockSpec((16, 16), lambda i: (0, 0)),
        out_shape=jax.ShapeDtypeStruct((16, 16), jnp.float32),
    )(xt, nd)

    cnt = acc[:, 0]
    s1 = acc[:, 1:4]
    s2 = jnp.stack([acc[:, 4], acc[:, 5], acc[:, 6],
                    acc[:, 5], acc[:, 7], acc[:, 8],
                    acc[:, 6], acc[:, 8], acc[:, 9]], axis=1).reshape(16, 3, 3)
    denom = jnp.maximum(cnt, 1.0)
    cov = s2 - (s1[:, :, None] * s1[:, None, :]) / denom[:, None, None]
    _, _, vh = jnp.linalg.svd(cov)
    rn = vh[:, 2, :]
    sign = jnp.where(jnp.sum(rn * normals, axis=1) < 0, -1.0, 1.0)
    rn = rn * sign[:, None]
    centroid = s1 / denom[:, None]
    rd = jnp.sum(centroid * rn, axis=1)
    gate = (cnt >= 3.0).astype(jnp.float32)
    rf = jnp.concatenate(
        [rn, rd[:, None], gate[:, None], jnp.zeros((16, 3), jnp.float32)],
        axis=1)

    out_t = pl.pallas_call(
        _proj_kernel,
        grid=(grid,),
        in_specs=[pl.BlockSpec((3, block), lambda i: (0, i)),
                  pl.BlockSpec((16, 4), lambda i: (0, 0)),
                  pl.BlockSpec((16, 8), lambda i: (0, 0))],
        out_specs=pl.BlockSpec((3, block), lambda i: (0, i)),
        out_shape=jax.ShapeDtypeStruct((3, n_pts), jnp.float32),
    )(xt, nd, rf)
    return out_t.T
